# baseline (speedup 1.0000x reference)
import functools

import jax
import jax.numpy as jnp
from jax import lax
from jax.experimental import pallas as pl
from jax.experimental.pallas import tpu as pltpu
from jax.experimental.pallas import tpu_sc as plsc

B = 1024
L = 50
NUM_EDGES = 1000
EDGE_DIM = 128
DIRECTION = 8
DIR_DIM = 32
HID = 512
PRE_LEN = 5
LINK_OUT = NUM_EDGES * PRE_LEN
DIR_OUT = DIRECTION * PRE_LEN

N_TOK = B * L
_CHUNK = 80
_SEG = 400
_OH = 16
_KCAT = EDGE_DIM + _OH + HID

_TRANS_B = (((1,), (1,)), ((), ()))


def _sc_gather(link_emb, idx_link):
    info = plsc.get_sparse_core_info()
    nc, ns = info.num_cores, info.num_subcores
    nw = nc * ns
    per_w = N_TOK // nw
    n_seg = per_w // _SEG

    mesh = plsc.VectorSubcoreMesh(core_axis_name="c", subcore_axis_name="s")

    @functools.partial(
        pl.kernel,
        mesh=mesh,
        out_type=jax.ShapeDtypeStruct((N_TOK, EDGE_DIM), jnp.float32),
        scratch_types=[
            pltpu.VMEM((per_w,), jnp.int32),
            pltpu.VMEM((_SEG, EDGE_DIM), jnp.float32),
            pltpu.VMEM((_SEG, EDGE_DIM), jnp.float32),
            pltpu.SemaphoreType.DMA,
            pltpu.SemaphoreType.DMA,
            pltpu.SemaphoreType.DMA,
            pltpu.SemaphoreType.DMA,
        ],
    )
    def gather_k(link_hbm, il_hbm, out_l, il_v, rows0, rows1, g0, g1, w0, w1):
        wid = lax.axis_index("s") * nc + lax.axis_index("c")
        base = wid * per_w
        pltpu.sync_copy(il_hbm.at[pl.ds(base, per_w)], il_v)
        rows = (rows0, rows1)
        gsem = (g0, g1)
        wsem = (w0, w1)
        wb = [None, None]
        for seg in range(n_seg):
            bi = seg & 1
            if wb[bi] is not None:
                wb[bi].wait()
            cps = []
            for k in range(_SEG // _CHUNK):
                off = seg * _SEG + k * _CHUNK
                cps.append(pltpu.async_copy(
                    link_hbm.at[il_v.at[pl.ds(off, _CHUNK)]],
                    rows[bi].at[pl.ds(k * _CHUNK, _CHUNK)],
                    gsem[bi]))
            for c in cps:
                c.wait()
            wb[bi] = pltpu.async_copy(
                rows[bi], out_l.at[pl.ds(base + seg * _SEG, _SEG)], wsem[bi])
        wb[0].wait()
        wb[1].wait()

    return gather_k(link_emb, idx_link)


def _lstm_body(len_ref, bias_ref, wih_ref, whh_ref, dir128_ref,
               xl_ref, di_ref, out_ref, h_scr, c_scr, xcat_scr, wcat_scr):
    t = pl.program_id(0)

    @pl.when(t == 0)
    def _():
        h_scr[...] = jnp.zeros_like(h_scr)
        c_scr[...] = jnp.zeros_like(c_scr)
        xcat_scr[:, EDGE_DIM + _OH:] = jnp.zeros((B, HID), jnp.bfloat16)
        r = lax.broadcasted_iota(jnp.int32, (4 * HID, 1), 0)
        is_g = (r >= 2 * HID) & (r < 3 * HID)
        s = jnp.where(is_g, jnp.float32(1.0), jnp.float32(0.5))
        wcat_scr[:, :EDGE_DIM] = (
            wih_ref[:, :EDGE_DIM] * s).astype(jnp.bfloat16)
        pdw = lax.dot_general(wih_ref[:, EDGE_DIM:], dir128_ref[...],
                              _TRANS_B, preferred_element_type=jnp.float32)
        wcat_scr[:, EDGE_DIM:EDGE_DIM + _OH] = (
            (pdw + bias_ref[...]) * s).astype(jnp.bfloat16)
        wcat_scr[:, EDGE_DIM + _OH:] = (whh_ref[...] * s).astype(jnp.bfloat16)

    xcat_scr[:, :EDGE_DIM] = xl_ref[0].astype(jnp.bfloat16)
    xcat_scr[:, EDGE_DIM:EDGE_DIM + _OH] = (
        lax.broadcasted_iota(jnp.int32, (B, _OH), 1)
        == di_ref[0]).astype(jnp.bfloat16)
    gates = lax.dot_general(xcat_scr[...], wcat_scr[...], _TRANS_B,
                            preferred_element_type=jnp.float32)
    i = 0.5 * jnp.tanh(gates[:, 0:HID]) + 0.5
    f = 0.5 * jnp.tanh(gates[:, HID:2 * HID]) + 0.5
    g = jnp.tanh(gates[:, 2 * HID:3 * HID])
    o = 0.5 * jnp.tanh(gates[:, 3 * HID:4 * HID]) + 0.5
    c_new = f * c_scr[...] + i * g
    h_new = o * jnp.tanh(c_new)
    valid = t < len_ref[...]
    h = jnp.where(valid, h_new, h_scr[...])
    h_scr[...] = h
    c_scr[...] = jnp.where(valid, c_new, c_scr[...])
    xcat_scr[:, EDGE_DIM + _OH:] = h.astype(jnp.bfloat16)

    @pl.when(t == L - 1)
    def _():
        out_ref[...] = h_scr[...]


def _run_lstm(len_i32, bias, W_ih, W_hh, dir128, xl, di):
    return pl.pallas_call(
        _lstm_body,
        grid=(L,),
        in_specs=[
            pl.BlockSpec((B, 1), lambda t: (0, 0)),
            pl.BlockSpec((4 * HID, 1), lambda t: (0, 0)),
            pl.BlockSpec((4 * HID, EDGE_DIM + DIR_DIM), lambda t: (0, 0)),
            pl.BlockSpec((4 * HID, HID), lambda t: (0, 0)),
            pl.BlockSpec((_OH, DIR_DIM), lambda t: (0, 0)),
            pl.BlockSpec((1, B, EDGE_DIM), lambda t: (t, 0, 0)),
            pl.BlockSpec((1, B, 1), lambda t: (t, 0, 0)),
        ],
        out_specs=pl.BlockSpec((B, HID), lambda t: (0, 0)),
        out_shape=jax.ShapeDtypeStruct((B, HID), jnp.float32),
        scratch_shapes=[
            pltpu.VMEM((B, HID), jnp.float32),
            pltpu.VMEM((B, HID), jnp.float32),
            pltpu.VMEM((B, _KCAT), jnp.bfloat16),
            pltpu.VMEM((4 * HID, _KCAT), jnp.bfloat16),
        ],
        compiler_params=pltpu.CompilerParams(
            dimension_semantics=("arbitrary",)),
    )(len_i32, bias, W_ih, W_hh, dir128, xl, di)


def _proj_body(h_ref, w_ref, b_ref, o_ref):
    h_bf = h_ref[...].astype(jnp.bfloat16)
    w_bf = w_ref[...].astype(jnp.bfloat16)
    o_ref[...] = lax.dot_general(
        h_bf, w_bf, _TRANS_B,
        preferred_element_type=jnp.float32) + b_ref[...]


def _run_proj(h, w, bias, n_out, bn):
    nb_n = (n_out + bn - 1) // bn
    nb_m = B // 256
    return pl.pallas_call(
        _proj_body,
        grid=(nb_n, nb_m),
        in_specs=[
            pl.BlockSpec((256, HID), lambda n, m: (m, 0)),
            pl.BlockSpec((bn, HID), lambda n, m: (n, 0)),
            pl.BlockSpec((1, bn), lambda n, m: (0, n)),
        ],
        out_specs=pl.BlockSpec((256, bn), lambda n, m: (m, n)),
        out_shape=jax.ShapeDtypeStruct((B, n_out), jnp.float32),
        compiler_params=pltpu.CompilerParams(
            dimension_semantics=("arbitrary", "arbitrary")),
    )(h, w, bias)


def kernel(inputs, directions, mask, link_emb, dir_emb, W_ih, W_hh,
           b_ih, b_hh, W_link, b_link, W_dir, b_dir):
    idx_l = inputs.astype(jnp.int32).T.reshape(-1)
    di = directions.astype(jnp.int32).T.reshape(L, B, 1)
    dir128 = jnp.pad(dir_emb, ((0, _OH - DIRECTION - 1), (0, 0)))
    bias = (b_ih + b_hh).reshape(4 * HID, 1)
    len_i32 = mask.astype(jnp.int32).reshape(B, 1)

    lrows = _sc_gather(link_emb, idx_l)
    xl = lrows.reshape(L, B, EDGE_DIM)
    h_n = _run_lstm(len_i32, bias, W_ih, W_hh, dir128, xl, di)

    pred = _run_proj(h_n, W_link, b_link.reshape(1, LINK_OUT),
                     LINK_OUT, 1280)
    pred_d = _run_proj(h_n, W_dir, b_dir.reshape(1, DIR_OUT),
                       DIR_OUT, DIR_OUT)
    return (pred, pred_d)


# speedup vs baseline: 1.0246x; 1.0246x over previous
import functools

import jax
import jax.numpy as jnp
from jax import lax
from jax.experimental import pallas as pl
from jax.experimental.pallas import tpu as pltpu
from jax.experimental.pallas import tpu_sc as plsc

B = 1024
L = 50
NUM_EDGES = 1000
EDGE_DIM = 128
DIRECTION = 8
DIR_DIM = 32
HID = 512
PRE_LEN = 5
LINK_OUT = NUM_EDGES * PRE_LEN
DIR_OUT = DIRECTION * PRE_LEN

N_TOK = B * L
_CHUNK = 80
_SEG = 400
_OH = 128
_KCAT = EDGE_DIM + _OH + HID

_TRANS_B = (((1,), (1,)), ((), ()))


def _sc_gather(link_emb, idx_link):
    info = plsc.get_sparse_core_info()
    nc, ns = info.num_cores, info.num_subcores
    nw = nc * ns
    per_w = N_TOK // nw
    n_seg = per_w // _SEG

    mesh = plsc.VectorSubcoreMesh(core_axis_name="c", subcore_axis_name="s")

    @functools.partial(
        pl.kernel,
        mesh=mesh,
        out_type=jax.ShapeDtypeStruct((N_TOK, EDGE_DIM), jnp.float32),
        scratch_types=[
            pltpu.VMEM((per_w,), jnp.int32),
            pltpu.VMEM((_SEG, EDGE_DIM), jnp.float32),
            pltpu.VMEM((_SEG, EDGE_DIM), jnp.float32),
            pltpu.SemaphoreType.DMA,
            pltpu.SemaphoreType.DMA,
            pltpu.SemaphoreType.DMA,
            pltpu.SemaphoreType.DMA,
        ],
    )
    def gather_k(link_hbm, il_hbm, out_l, il_v, rows0, rows1, g0, g1, w0, w1):
        wid = lax.axis_index("s") * nc + lax.axis_index("c")
        base = wid * per_w
        pltpu.sync_copy(il_hbm.at[pl.ds(base, per_w)], il_v)
        rows = (rows0, rows1)
        gsem = (g0, g1)
        wsem = (w0, w1)
        wb = [None, None]
        for seg in range(n_seg):
            bi = seg & 1
            if wb[bi] is not None:
                wb[bi].wait()
            cps = []
            for k in range(_SEG // _CHUNK):
                off = seg * _SEG + k * _CHUNK
                cps.append(pltpu.async_copy(
                    link_hbm.at[il_v.at[pl.ds(off, _CHUNK)]],
                    rows[bi].at[pl.ds(k * _CHUNK, _CHUNK)],
                    gsem[bi]))
            for c in cps:
                c.wait()
            wb[bi] = pltpu.async_copy(
                rows[bi], out_l.at[pl.ds(base + seg * _SEG, _SEG)], wsem[bi])
        wb[0].wait()
        wb[1].wait()

    return gather_k(link_emb, idx_link)


def _lstm_body(len_ref, bias_ref, wih_ref, whh_ref, dir128_ref,
               xl_ref, di_ref, out_ref, h_scr, c_scr, xcat_scr, wcat_scr):
    t = pl.program_id(0)

    @pl.when(t == 0)
    def _():
        h_scr[...] = jnp.zeros_like(h_scr)
        c_scr[...] = jnp.zeros_like(c_scr)
        xcat_scr[:, EDGE_DIM + _OH:] = jnp.zeros((B, HID), jnp.bfloat16)
        r = lax.broadcasted_iota(jnp.int32, (4 * HID, 1), 0)
        is_g = (r >= 2 * HID) & (r < 3 * HID)
        s = jnp.where(is_g, jnp.float32(1.0), jnp.float32(0.5))
        wcat_scr[:, :EDGE_DIM] = (
            wih_ref[:, :EDGE_DIM] * s).astype(jnp.bfloat16)
        pdw = lax.dot_general(wih_ref[:, EDGE_DIM:], dir128_ref[...],
                              _TRANS_B, preferred_element_type=jnp.float32)
        wcat_scr[:, EDGE_DIM:EDGE_DIM + _OH] = (
            (pdw + bias_ref[...]) * s).astype(jnp.bfloat16)
        wcat_scr[:, EDGE_DIM + _OH:] = (whh_ref[...] * s).astype(jnp.bfloat16)

    xcat_scr[:, :EDGE_DIM] = xl_ref[0].astype(jnp.bfloat16)
    xcat_scr[:, EDGE_DIM:EDGE_DIM + _OH] = (
        lax.broadcasted_iota(jnp.int32, (B, _OH), 1)
        == di_ref[0]).astype(jnp.bfloat16)
    gates = lax.dot_general(xcat_scr[...], wcat_scr[...], _TRANS_B,
                            preferred_element_type=jnp.float32)
    i = 0.5 * jnp.tanh(gates[:, 0:HID]) + 0.5
    f = 0.5 * jnp.tanh(gates[:, HID:2 * HID]) + 0.5
    g = jnp.tanh(gates[:, 2 * HID:3 * HID])
    o = 0.5 * jnp.tanh(gates[:, 3 * HID:4 * HID]) + 0.5
    c_new = f * c_scr[...] + i * g
    h_new = o * jnp.tanh(c_new)
    valid = t < len_ref[...]
    h = jnp.where(valid, h_new, h_scr[...])
    h_scr[...] = h
    c_scr[...] = jnp.where(valid, c_new, c_scr[...])
    xcat_scr[:, EDGE_DIM + _OH:] = h.astype(jnp.bfloat16)

    @pl.when(t == L - 1)
    def _():
        out_ref[...] = h_scr[...]


def _run_lstm(len_i32, bias, W_ih, W_hh, dir128, xl, di):
    return pl.pallas_call(
        _lstm_body,
        grid=(L,),
        in_specs=[
            pl.BlockSpec((B, 1), lambda t: (0, 0)),
            pl.BlockSpec((4 * HID, 1), lambda t: (0, 0)),
            pl.BlockSpec((4 * HID, EDGE_DIM + DIR_DIM), lambda t: (0, 0)),
            pl.BlockSpec((4 * HID, HID), lambda t: (0, 0)),
            pl.BlockSpec((_OH, DIR_DIM), lambda t: (0, 0)),
            pl.BlockSpec((1, B, EDGE_DIM), lambda t: (t, 0, 0)),
            pl.BlockSpec((1, B, 1), lambda t: (t, 0, 0)),
        ],
        out_specs=pl.BlockSpec((B, HID), lambda t: (0, 0)),
        out_shape=jax.ShapeDtypeStruct((B, HID), jnp.float32),
        scratch_shapes=[
            pltpu.VMEM((B, HID), jnp.float32),
            pltpu.VMEM((B, HID), jnp.float32),
            pltpu.VMEM((B, _KCAT), jnp.bfloat16),
            pltpu.VMEM((4 * HID, _KCAT), jnp.bfloat16),
        ],
        compiler_params=pltpu.CompilerParams(
            dimension_semantics=("arbitrary",)),
    )(len_i32, bias, W_ih, W_hh, dir128, xl, di)


def _proj_body(h_ref, w_ref, b_ref, o_ref):
    h_bf = h_ref[...].astype(jnp.bfloat16)
    w_bf = w_ref[...].astype(jnp.bfloat16)
    o_ref[...] = lax.dot_general(
        h_bf, w_bf, _TRANS_B,
        preferred_element_type=jnp.float32) + b_ref[...]


def _run_proj(h, w, bias, n_out, bn):
    nb_n = (n_out + bn - 1) // bn
    nb_m = B // 256
    return pl.pallas_call(
        _proj_body,
        grid=(nb_n, nb_m),
        in_specs=[
            pl.BlockSpec((256, HID), lambda n, m: (m, 0)),
            pl.BlockSpec((bn, HID), lambda n, m: (n, 0)),
            pl.BlockSpec((1, bn), lambda n, m: (0, n)),
        ],
        out_specs=pl.BlockSpec((256, bn), lambda n, m: (m, n)),
        out_shape=jax.ShapeDtypeStruct((B, n_out), jnp.float32),
        compiler_params=pltpu.CompilerParams(
            dimension_semantics=("arbitrary", "arbitrary")),
    )(h, w, bias)


def kernel(inputs, directions, mask, link_emb, dir_emb, W_ih, W_hh,
           b_ih, b_hh, W_link, b_link, W_dir, b_dir):
    idx_l = inputs.astype(jnp.int32).T.reshape(-1)
    di = directions.astype(jnp.int32).T.reshape(L, B, 1)
    dir128 = jnp.pad(dir_emb, ((0, _OH - DIRECTION - 1), (0, 0)))
    bias = (b_ih + b_hh).reshape(4 * HID, 1)
    len_i32 = mask.astype(jnp.int32).reshape(B, 1)

    lrows = _sc_gather(link_emb, idx_l)
    xl = lrows.reshape(L, B, EDGE_DIM)
    h_n = _run_lstm(len_i32, bias, W_ih, W_hh, dir128, xl, di)

    pred = _run_proj(h_n, W_link, b_link.reshape(1, LINK_OUT),
                     LINK_OUT, 1280)
    pred_d = _run_proj(h_n, W_dir, b_dir.reshape(1, DIR_OUT),
                       DIR_OUT, DIR_OUT)
    return (pred, pred_d)
